# baseline (speedup 1.0000x reference)
import jax
import jax.numpy as jnp
from jax.experimental import pallas as pl
from jax.experimental.pallas import tpu as pltpu

_B, _S, _E, _D, _HID, _R = 4, 2048, 8, 1024, 1024, 16
_M0, _M1 = 32, 32
_TILE = 1024
_S_TILES = _S // _TILE


def _moe_kernel(x_ref, w1_ref, b1_ref, w2_ref, b2_ref,
                c0_ref, c1_ref, c2_ref, c3_ref, o_ref,
                hsum_ref, w1b_ref, mc0_ref, mc1_ref, mc2_ref, mc3_ref,
                a0_ref, a3_ref):
    ph = pl.program_id(0)
    b = pl.program_id(1)
    s = pl.program_id(2)

    @pl.when(jnp.logical_and(ph == 0, jnp.logical_and(b == 0, s == 0)))
    def _cast_w1():
        w1b_ref[...] = w1_ref[...].astype(jnp.bfloat16)

    @pl.when(ph == 0)
    def _router():
        @pl.when(s == 0)
        def _init():
            hsum_ref[...] = jnp.zeros_like(hsum_ref)

        x = x_ref[0].astype(jnp.bfloat16)
        h = jnp.dot(x, w1b_ref[...], preferred_element_type=jnp.float32)
        h = jnp.maximum(h + b1_ref[...], 0.0).astype(jnp.bfloat16)
        ones = jnp.ones((1, _TILE), jnp.bfloat16)
        hsum_ref[...] += jnp.dot(ones, h, preferred_element_type=jnp.float32)

        @pl.when(s == _S_TILES - 1)
        def _epilogue():
            hi = jax.lax.Precision.HIGHEST
            hbar = hsum_ref[...] * (1.0 / _S)
            logits = jax.lax.dot_general(
                hbar, w2_ref[...], (((1,), (0,)), ((), ())),
                preferred_element_type=jnp.float32,
                precision=hi) + b2_ref[...]
            m1 = jnp.max(logits, axis=1, keepdims=True)
            is_max = logits == m1
            nmax = jnp.sum(is_max.astype(jnp.float32), axis=1, keepdims=True)
            second = jnp.max(jnp.where(is_max, -jnp.inf, logits), axis=1,
                             keepdims=True)
            thresh = jnp.where(nmax >= 2.0, m1, second)
            mask = logits >= thresh
            ex = jnp.where(mask, jnp.exp(logits - m1), 0.0)
            gates = ex / jnp.sum(ex, axis=1, keepdims=True)
            lane = jax.lax.broadcasted_iota(jnp.int32, (1, _E), 1)
            mc0 = jnp.zeros((_M0, _R), jnp.float32)
            mc1 = jnp.zeros((_M1 * _R, _R), jnp.float32)
            mc2 = jnp.zeros((_R, _M0 * _R), jnp.float32)
            mc3 = jnp.zeros((_R, _M1), jnp.float32)
            for e in range(_E):
                g = jnp.sum(jnp.where(lane == e, gates, 0.0), axis=1,
                            keepdims=True)
                g = g.astype(jnp.bfloat16).astype(jnp.float32)
                mc0 = mc0 + g * c0_ref[e]
                mc1 = mc1 + g * c1_ref[e]
                mc2 = mc2 + g * c2_ref[e]
                mc3 = mc3 + g * c3_ref[e]
            mc0_ref[b] = mc0.astype(jnp.bfloat16)
            mc1_ref[b] = mc1.astype(jnp.bfloat16)
            mc2_ref[b] = mc2.astype(jnp.bfloat16)
            mc3_ref[b] = mc3.astype(jnp.bfloat16)

    @pl.when(ph == 1)
    def _apply():
        @pl.when(s == 0)
        def _build():
            bm = jnp.concatenate([mc0_ref[b]] * 16, axis=0)
            bm = jnp.concatenate([bm] * 16, axis=1)
            ri = jax.lax.broadcasted_iota(jnp.int32, (512, 256), 0) // _M0
            ci = jax.lax.broadcasted_iota(jnp.int32, (512, 256), 1) // _R
            a0_ref[...] = jnp.where(ri == ci, bm, jnp.bfloat16(0))
            cmm = jnp.concatenate([mc3_ref[b]] * 8, axis=0)
            cmm = jnp.concatenate([cmm] * 8, axis=1)
            rj = jax.lax.broadcasted_iota(jnp.int32, (128, 256), 0) // _R
            cj = jax.lax.broadcasted_iota(jnp.int32, (128, 256), 1) // _M1
            a3_ref[...] = jnp.where(rj == cj, cmm, jnp.bfloat16(0))

        x = x_ref[0].astype(jnp.bfloat16)
        t1h = [jnp.dot(x[:, 512 * k:512 * (k + 1)], a0_ref[...],
                       preferred_element_type=jnp.float32).astype(jnp.bfloat16)
               for k in range(2)]
        t2 = (jnp.dot(t1h[0], mc1_ref[b, :256],
                      preferred_element_type=jnp.float32) +
              jnp.dot(t1h[1], mc1_ref[b, 256:],
                      preferred_element_type=jnp.float32))
        t3 = jnp.dot(t2.astype(jnp.bfloat16), mc2_ref[b],
                     preferred_element_type=jnp.float32)
        t3 = t3.astype(jnp.bfloat16)
        for k in range(4):
            o_ref[0, :, 256 * k:256 * (k + 1)] = jnp.dot(
                t3[:, 128 * k:128 * (k + 1)], a3_ref[...],
                preferred_element_type=jnp.float32)


def kernel(X, W_expand, b_expand, W_proj, b_proj, core0, core1, core2, core3):
    b1 = b_expand.reshape(1, _HID)
    w2 = W_proj.astype(jnp.bfloat16).astype(jnp.float32)
    b2 = b_proj.reshape(1, _E)
    f32 = jnp.float32
    bf16 = jnp.bfloat16
    c0 = core0.reshape(_E, _M0, _R).astype(bf16).astype(f32)
    c1 = (core1.transpose(0, 2, 1, 3).reshape(_E, _M1 * _R, _R)
          .astype(bf16).astype(f32))
    c2 = core2.reshape(_E, _R, _M0 * _R).astype(bf16).astype(f32)
    c3 = core3.reshape(_E, _R, _M1).astype(bf16).astype(f32)

    Z = pl.pallas_call(
        _moe_kernel,
        grid=(2, _B, _S_TILES),
        in_specs=[
            pl.BlockSpec((1, _TILE, _D), lambda p, b, s: (b, s, 0)),
            pl.BlockSpec((_HID, _D), lambda p, b, s: (0, 0)),
            pl.BlockSpec((1, _HID), lambda p, b, s: (0, 0)),
            pl.BlockSpec((_HID, _E), lambda p, b, s: (0, 0)),
            pl.BlockSpec((1, _E), lambda p, b, s: (0, 0)),
            pl.BlockSpec(c0.shape, lambda p, b, s: (0, 0, 0)),
            pl.BlockSpec(c1.shape, lambda p, b, s: (0, 0, 0)),
            pl.BlockSpec(c2.shape, lambda p, b, s: (0, 0, 0)),
            pl.BlockSpec(c3.shape, lambda p, b, s: (0, 0, 0)),
        ],
        out_specs=pl.BlockSpec((1, _TILE, _D),
                               lambda p, b, s: (p * b, p * s, 0)),
        out_shape=jax.ShapeDtypeStruct((_B, _S, _D), jnp.float32),
        scratch_shapes=[
            pltpu.VMEM((1, _HID), jnp.float32),
            pltpu.VMEM((_HID, _D), bf16),
            pltpu.VMEM((_B, _M0, _R), bf16),
            pltpu.VMEM((_B, _M1 * _R, _R), bf16),
            pltpu.VMEM((_B, _R, _M0 * _R), bf16),
            pltpu.VMEM((_B, _R, _M1), bf16),
            pltpu.VMEM((512, 256), bf16),
            pltpu.VMEM((128, 256), bf16),
        ],
    )(X, W_expand, b1, w2, b2, c0, c1, c2, c3)
    return Z


# speedup vs baseline: 1.0306x; 1.0306x over previous
import jax
import jax.numpy as jnp
from jax.experimental import pallas as pl
from jax.experimental.pallas import tpu as pltpu

_B, _S, _E, _D, _HID, _R = 4, 2048, 8, 1024, 1024, 16
_M0, _M1 = 32, 32
_TILE = 2048
_S_TILES = _S // _TILE


def _moe_kernel(x_ref, w1_ref, b1_ref, w2_ref, b2_ref,
                c0_ref, c1_ref, c2_ref, c3_ref, o_ref,
                hsum_ref, w1b_ref, mc0_ref, mc1_ref, mc2_ref, mc3_ref,
                a0_ref, a3_ref):
    ph = pl.program_id(0)
    b = pl.program_id(1)
    s = pl.program_id(2)

    @pl.when(jnp.logical_and(ph == 0, jnp.logical_and(b == 0, s == 0)))
    def _cast_w1():
        w1b_ref[...] = w1_ref[...].astype(jnp.bfloat16)

    @pl.when(ph == 0)
    def _router():
        @pl.when(s == 0)
        def _init():
            hsum_ref[...] = jnp.zeros_like(hsum_ref)

        x = x_ref[0].astype(jnp.bfloat16)
        h = jnp.dot(x, w1b_ref[...], preferred_element_type=jnp.float32)
        h = jnp.maximum(h + b1_ref[...], 0.0).astype(jnp.bfloat16)
        ones = jnp.ones((1, _TILE), jnp.bfloat16)
        hsum_ref[...] += jnp.dot(ones, h, preferred_element_type=jnp.float32)

        @pl.when(s == _S_TILES - 1)
        def _epilogue():
            hi = jax.lax.Precision.HIGHEST
            hbar = hsum_ref[...] * (1.0 / _S)
            logits = jax.lax.dot_general(
                hbar, w2_ref[...], (((1,), (0,)), ((), ())),
                preferred_element_type=jnp.float32,
                precision=hi) + b2_ref[...]
            m1 = jnp.max(logits, axis=1, keepdims=True)
            is_max = logits == m1
            nmax = jnp.sum(is_max.astype(jnp.float32), axis=1, keepdims=True)
            second = jnp.max(jnp.where(is_max, -jnp.inf, logits), axis=1,
                             keepdims=True)
            thresh = jnp.where(nmax >= 2.0, m1, second)
            mask = logits >= thresh
            ex = jnp.where(mask, jnp.exp(logits - m1), 0.0)
            gates = ex / jnp.sum(ex, axis=1, keepdims=True)
            lane = jax.lax.broadcasted_iota(jnp.int32, (1, _E), 1)
            mc0 = jnp.zeros((_M0, _R), jnp.float32)
            mc1 = jnp.zeros((_M1 * _R, _R), jnp.float32)
            mc2 = jnp.zeros((_R, _M0 * _R), jnp.float32)
            mc3 = jnp.zeros((_R, _M1), jnp.float32)
            for e in range(_E):
                g = jnp.sum(jnp.where(lane == e, gates, 0.0), axis=1,
                            keepdims=True)
                g = g.astype(jnp.bfloat16).astype(jnp.float32)
                mc0 = mc0 + g * c0_ref[e]
                mc1 = mc1 + g * c1_ref[e]
                mc2 = mc2 + g * c2_ref[e]
                mc3 = mc3 + g * c3_ref[e]
            mc0_ref[b] = mc0.astype(jnp.bfloat16)
            mc1_ref[b] = mc1.astype(jnp.bfloat16)
            mc2_ref[b] = mc2.astype(jnp.bfloat16)
            mc3_ref[b] = mc3.astype(jnp.bfloat16)

    @pl.when(ph == 1)
    def _apply():
        @pl.when(s == 0)
        def _build():
            bm = jnp.concatenate([mc0_ref[b]] * 16, axis=0)
            bm = jnp.concatenate([bm] * 16, axis=1)
            ri = jax.lax.broadcasted_iota(jnp.int32, (512, 256), 0) // _M0
            ci = jax.lax.broadcasted_iota(jnp.int32, (512, 256), 1) // _R
            a0_ref[...] = jnp.where(ri == ci, bm, jnp.bfloat16(0))
            cmm = jnp.concatenate([mc3_ref[b]] * 8, axis=0)
            cmm = jnp.concatenate([cmm] * 8, axis=1)
            rj = jax.lax.broadcasted_iota(jnp.int32, (128, 256), 0) // _R
            cj = jax.lax.broadcasted_iota(jnp.int32, (128, 256), 1) // _M1
            a3_ref[...] = jnp.where(rj == cj, cmm, jnp.bfloat16(0))

        x = x_ref[0].astype(jnp.bfloat16)
        t1h = [jnp.dot(x[:, 512 * k:512 * (k + 1)], a0_ref[...],
                       preferred_element_type=jnp.float32).astype(jnp.bfloat16)
               for k in range(2)]
        t2 = (jnp.dot(t1h[0], mc1_ref[b, :256],
                      preferred_element_type=jnp.float32) +
              jnp.dot(t1h[1], mc1_ref[b, 256:],
                      preferred_element_type=jnp.float32))
        t3 = jnp.dot(t2.astype(jnp.bfloat16), mc2_ref[b],
                     preferred_element_type=jnp.float32)
        t3 = t3.astype(jnp.bfloat16)
        for k in range(4):
            o_ref[0, :, 256 * k:256 * (k + 1)] = jnp.dot(
                t3[:, 128 * k:128 * (k + 1)], a3_ref[...],
                preferred_element_type=jnp.float32)


def kernel(X, W_expand, b_expand, W_proj, b_proj, core0, core1, core2, core3):
    b1 = b_expand.reshape(1, _HID)
    w2 = W_proj.astype(jnp.bfloat16).astype(jnp.float32)
    b2 = b_proj.reshape(1, _E)
    f32 = jnp.float32
    bf16 = jnp.bfloat16
    c0 = core0.reshape(_E, _M0, _R).astype(bf16).astype(f32)
    c1 = (core1.transpose(0, 2, 1, 3).reshape(_E, _M1 * _R, _R)
          .astype(bf16).astype(f32))
    c2 = core2.reshape(_E, _R, _M0 * _R).astype(bf16).astype(f32)
    c3 = core3.reshape(_E, _R, _M1).astype(bf16).astype(f32)

    Z = pl.pallas_call(
        _moe_kernel,
        grid=(2, _B, _S_TILES),
        in_specs=[
            pl.BlockSpec((1, _TILE, _D), lambda p, b, s: (b, s, 0)),
            pl.BlockSpec((_HID, _D), lambda p, b, s: (0, 0)),
            pl.BlockSpec((1, _HID), lambda p, b, s: (0, 0)),
            pl.BlockSpec((_HID, _E), lambda p, b, s: (0, 0)),
            pl.BlockSpec((1, _E), lambda p, b, s: (0, 0)),
            pl.BlockSpec(c0.shape, lambda p, b, s: (0, 0, 0)),
            pl.BlockSpec(c1.shape, lambda p, b, s: (0, 0, 0)),
            pl.BlockSpec(c2.shape, lambda p, b, s: (0, 0, 0)),
            pl.BlockSpec(c3.shape, lambda p, b, s: (0, 0, 0)),
        ],
        out_specs=pl.BlockSpec((1, _TILE, _D),
                               lambda p, b, s: (p * b, p * s, 0)),
        out_shape=jax.ShapeDtypeStruct((_B, _S, _D), jnp.float32),
        scratch_shapes=[
            pltpu.VMEM((1, _HID), jnp.float32),
            pltpu.VMEM((_HID, _D), bf16),
            pltpu.VMEM((_B, _M0, _R), bf16),
            pltpu.VMEM((_B, _M1 * _R, _R), bf16),
            pltpu.VMEM((_B, _R, _M0 * _R), bf16),
            pltpu.VMEM((_B, _R, _M1), bf16),
            pltpu.VMEM((512, 256), bf16),
            pltpu.VMEM((128, 256), bf16),
        ],
    )(X, W_expand, b1, w2, b2, c0, c1, c2, c3)
    return Z
